# baseline (speedup 1.0000x reference)
import jax
import jax.numpy as jnp
from jax.experimental import pallas as pl
from jax.experimental.pallas import tpu as pltpu

NUM_PARTITIONS = 8
TBLK = 128
NSLOT = 4


def _mask_bcast_kernel(idx_ref, x_ref, out_hbm, scratch, sems):
    i = pl.program_id(0)
    n = pl.num_programs(0)
    slot = jax.lax.rem(i, NSLOT)
    K = idx_ref.shape[1]

    def wait_slot(s, step):
        pltpu.make_async_copy(
            scratch.at[s],
            out_hbm.at[pl.ds(step * TBLK, TBLK)],
            sems.at[s],
        ).wait()

    @pl.when(i >= NSLOT)
    def _():
        wait_slot(slot, i - NSLOT)

    piota = jax.lax.broadcasted_iota(
        jnp.int32, (TBLK, NUM_PARTITIONS, 1), 1)
    m = idx_ref[:, 0:1, :] == piota
    for k in range(1, K):
        m = m | (idx_ref[:, k:k + 1, :] == piota)
    blk = jnp.where(m, x_ref[...], 0.0)

    for s in range(NSLOT):
        @pl.when(slot == s)
        def _(s=s):
            scratch[s] = blk

    pltpu.make_async_copy(
        scratch.at[slot],
        out_hbm.at[pl.ds(i * TBLK, TBLK)],
        sems.at[slot],
    ).start()

    @pl.when(i == n - 1)
    def _():
        for d in range(min(NSLOT - 1, n - 1), -1, -1):
            wait_slot(jax.lax.rem(i - d + NSLOT, NSLOT), i - d)


def kernel(x, partition_indices):
    B, S, D = x.shape
    T = B * S
    K = partition_indices.shape[-1]
    xf = x.reshape(T, 1, D)
    idx = partition_indices.reshape(T, K, 1).astype(jnp.int32)

    out = pl.pallas_call(
        _mask_bcast_kernel,
        grid=(T // TBLK,),
        in_specs=[
            pl.BlockSpec((TBLK, K, 1), lambda i: (i, 0, 0)),
            pl.BlockSpec((TBLK, 1, D), lambda i: (i, 0, 0)),
        ],
        out_specs=pl.BlockSpec(memory_space=pl.ANY),
        out_shape=jax.ShapeDtypeStruct((T, NUM_PARTITIONS, D), x.dtype),
        scratch_shapes=[
            pltpu.VMEM((NSLOT, TBLK, NUM_PARTITIONS, D), x.dtype),
            pltpu.SemaphoreType.DMA((NSLOT,)),
        ],
    )(idx, xf)
    return out.reshape(B, S, NUM_PARTITIONS, D)


# speedup vs baseline: 1.4243x; 1.4243x over previous
import functools

import jax
import jax.numpy as jnp
from jax import lax
from jax.experimental import pallas as pl
from jax.experimental.pallas import tpu as pltpu
from jax.experimental.pallas import tpu_sc as plsc

NUM_PARTITIONS = 8
P = NUM_PARTITIONS
NW = 32
CHUNK = 16


def _sc_body(Tw, K, D, x_hbm, idx_hbm, out_hbm,
             xbuf, idxbuf, zrow, xsem, wsem):
    nchunks = Tw // CHUNK
    wid = lax.axis_index("s") * 2 + lax.axis_index("c")
    tbase = wid * Tw

    for v in range(D // 16):
        zrow[0, pl.ds(v * 16, 16)] = jnp.zeros((16,), jnp.float32)

    pltpu.sync_copy(idx_hbm.at[pl.ds(tbase * K, Tw * K)],
                    idxbuf.at[pl.ds(0, Tw * K)])

    def load_chunk(ci):
        pltpu.make_async_copy(
            x_hbm.at[pl.ds(tbase + ci * CHUNK, CHUNK)],
            xbuf.at[ci % 2],
            xsem,
        ).start()

    def wait_chunk(ci):
        pltpu.make_async_copy(
            x_hbm.at[pl.ds(tbase + ci * CHUNK, CHUNK)],
            xbuf.at[ci % 2],
            xsem,
        ).wait()

    def drain_rows(n):
        def body(j, carry):
            pltpu.make_async_copy(
                x_hbm.at[pl.ds(0, 1)], zrow, wsem).wait()
            return carry
        lax.fori_loop(0, n, body, 0)

    load_chunk(0)
    for ci in range(nchunks):
        wait_chunk(ci)
        slot = ci % 2

        def tok_body(t, carry, ci=ci, slot=slot):
            tloc = ci * CHUNK + t
            g = tbase + tloc
            iv = idxbuf[pl.ds(tloc * K, 16)]
            i0 = iv[0]
            i1 = iv[1] if K > 1 else i0
            for p in range(NUM_PARTITIONS):
                sel = (i0 == p) | (i1 == p)
                dst = out_hbm.at[pl.ds(g * P + p, 1)]

                @pl.when(sel)
                def _():
                    pltpu.make_async_copy(
                        xbuf.at[slot, pl.ds(t, 1)], dst, wsem).start()

                @pl.when(jnp.logical_not(sel))
                def _():
                    pltpu.make_async_copy(zrow, dst, wsem).start()
            return carry

        lax.fori_loop(0, CHUNK, tok_body, 0)

        if ci + 1 < nchunks:
            if ci >= 1:
                drain_rows(CHUNK * P)
            load_chunk(ci + 1)
    drain_rows(min(2, nchunks) * CHUNK * P)


def kernel(x, partition_indices):
    B, S, D = x.shape
    T = B * S
    K = partition_indices.shape[-1]
    Tw = T // NW
    x2d = x.reshape(T, D)
    idxf = partition_indices.reshape(T * K).astype(jnp.int32)

    body = functools.partial(_sc_body, Tw, K, D)
    out = pl.kernel(
        body,
        out_type=jax.ShapeDtypeStruct((T * P, D), jnp.float32),
        mesh=plsc.VectorSubcoreMesh(core_axis_name="c", subcore_axis_name="s"),
        scratch_types=[
            pltpu.VMEM((2, CHUNK, D), jnp.float32),
            pltpu.VMEM((Tw * K + 16,), jnp.int32),
            pltpu.VMEM((1, D), jnp.float32),
            pltpu.SemaphoreType.DMA,
            pltpu.SemaphoreType.DMA,
        ],
    )(x2d, idxf)
    return out.reshape(B, S, P, D)
